# baseline (speedup 1.0000x reference)
import functools

import jax
import jax.numpy as jnp
from jax import lax
from jax.experimental import pallas as pl
from jax.experimental.pallas import tpu as pltpu
from jax.experimental.pallas import tpu_sc as plsc

N = 10000
E = 320000
ET = E + N
NPAD = 10240
G = 128
NC = 2
NS = 16
NW = NC * NS
CH = 88
EPAD = NW * CH * G
EROWS = EPAD // G
STR = NPAD // NS

D1, W1ROW = 64, 72
D2, W2ROW = 2, 16

_SC_PARAMS = pltpu.CompilerParams(needs_layout_passes=False,
                                  use_tc_tiling_on_sc=False)
_MESH = plsc.VectorSubcoreMesh(core_axis_name="c", subcore_axis_name="s")


def _dense_pack(h, a_src, a_dst, wrow):
    n, d = h.shape
    a_s = jnp.sum(h * a_src, axis=1, keepdims=True)
    a_d = jnp.sum(h * a_dst, axis=1, keepdims=True)
    g = jnp.max(a_s) + jnp.max(a_d)
    g = jnp.where(g >= 0.0, g, 0.2 * g)
    colid = lax.broadcasted_iota(jnp.int32, (n, wrow - d), 1)
    pat = jnp.where(colid == 0, 1.0, 0.0).astype(jnp.float32)
    hp = jnp.concatenate([h, pat], axis=1)
    hp = jnp.concatenate([hp, jnp.zeros((NPAD - n, wrow), jnp.float32)], axis=0)
    asad = jnp.concatenate([a_s, a_d], axis=1)
    asad = jnp.concatenate([asad, jnp.zeros((NPAD - n, 2), jnp.float32)], axis=0)
    return hp, asad, jnp.full((1, 16), g, jnp.float32)


def _dense1_body(x_ref, w_ref, asrc_ref, adst_ref, hpad_ref, asad_ref, g_ref):
    h = jnp.dot(x_ref[:], w_ref[:], preferred_element_type=jnp.float32)
    hp, asad, g = _dense_pack(h, asrc_ref[:], adst_ref[:], W1ROW)
    hpad_ref[:] = hp
    asad_ref[:] = asad
    g_ref[:] = g


def _dense2_body(p_ref, w_ref, asrc_ref, adst_ref, b1_ref,
                 hpad_ref, asad_ref, g_ref):
    p = p_ref[0] + p_ref[1]
    feat = p[0:N, 0:D1]
    den = p[0:N, D1:D1 + 1]
    h1 = feat / (den + 1e-16) + b1_ref[:]
    h1 = jnp.where(h1 > 0.0, h1, jnp.exp(h1) - 1.0)
    h2 = jnp.dot(h1, w_ref[:], preferred_element_type=jnp.float32)
    hp, asad, g = _dense_pack(h2, asrc_ref[:], adst_ref[:], W2ROW)
    hpad_ref[:] = hp
    asad_ref[:] = asad
    g_ref[:] = g


def _final_body(p_ref, b2_ref, out_ref):
    p = p_ref[0] + p_ref[1]
    out_ref[:] = p[0:N, 0:D2] / (p[0:N, D2:D2 + 1] + 1e-16) + b2_ref[:]


def _make_edge_kernel(wrow, dcols):
    nscale = dcols // 16 if dcols % 16 == 0 else wrow // 16

    @functools.partial(
        pl.kernel,
        out_type=jax.ShapeDtypeStruct((NC, NPAD, wrow), jnp.float32),
        mesh=_MESH,
        compiler_params=_SC_PARAMS,
        scratch_types=[
            pltpu.VMEM((NPAD,), jnp.float32),
            pltpu.VMEM((NPAD,), jnp.float32),
            pltpu.VMEM((16,), jnp.float32),
            pltpu.VMEM((CH, G), jnp.int32),
            pltpu.VMEM((CH, G), jnp.int32),
            pltpu.VMEM((G,), jnp.float32),
            [pltpu.VMEM((G, wrow), jnp.float32) for _ in range(2)],
            [pltpu.SemaphoreType.DMA for _ in range(2)],
            pltpu.VMEM_SHARED((NPAD, wrow), jnp.float32),
        ],
    )
    def edge_kernel(hpad_hbm, as_hbm, ad_hbm, g_hbm, src_hbm, dst_hbm,
                    outp_hbm, as_v, ad_v, g_v, src_v, dst_v, ee_v, rows,
                    gsem, accum):
        cid = lax.axis_index("c")
        sid = lax.axis_index("s")
        wid = sid * NC + cid

        def zrow(r, carry):
            for kk in range(wrow // 16):
                rows[0][r, pl.ds(kk * 16, 16)] = jnp.zeros((16,), jnp.float32)
            return carry

        lax.fori_loop(0, G, zrow, 0)
        for i in range(STR // G):
            pltpu.sync_copy(rows[0], accum.at[pl.ds(sid * STR + i * G, G)])

        pltpu.sync_copy(as_hbm, as_v)
        pltpu.sync_copy(ad_hbm, ad_v)
        pltpu.sync_copy(g_hbm, g_v)
        pltpu.sync_copy(src_hbm.at[pl.ds(wid * CH, CH)], src_v)
        pltpu.sync_copy(dst_hbm.at[pl.ds(wid * CH, CH)], dst_v)
        plsc.subcore_barrier()

        gv = g_v[:]

        def gather_issue(c, b):
            pltpu.async_copy(hpad_hbm.at[src_v.at[c]], rows[b], gsem[b])

        def gather_wait(c, b):
            pltpu.make_async_copy(hpad_hbm.at[src_v.at[c]], rows[b],
                                  gsem[b]).wait()

        def process(c, b):
            f = c + 1
            bf = 1 - b

            @pl.when(f < CH)
            def _():
                gather_issue(f, bf)

            for kk in range(G // 16):
                sl = pl.ds(kk * 16, 16)
                e = (plsc.load_gather(as_v, [src_v[c, sl]]) +
                     plsc.load_gather(ad_v, [dst_v[c, sl]]))
                e = jnp.where(e >= 0.0, e, 0.2 * e)
                ee_v[sl] = jnp.exp(e - gv)

            gather_wait(c, b)

            if dcols % 16 == 0:
                ci = lax.iota(jnp.int32, 16)
                csp = jnp.full((16,), dcols, jnp.int32)
                for kk in range(G // 16):
                    plsc.store_scatter(rows[b], [ci + kk * 16, csp],
                                       ee_v[pl.ds(kk * 16, 16)])

            def row_body(i, rcarry):
                for u in range(8):
                    r = i * 8 + u
                    s = plsc.load_gather(ee_v, [jnp.full((16,), r, jnp.int32)])
                    for kk in range(nscale):
                        sl = pl.ds(kk * 16, 16)
                        rows[b][r, sl] = rows[b][r, sl] * s
                return rcarry

            lax.fori_loop(0, G // 8, row_body, 0)
            pltpu.sync_copy(rows[b], accum.at[dst_v.at[c]], add=True)

        gather_issue(0, 0)

        def chunk_body(c2, carry):
            for u in range(2):
                process(c2 * 2 + u, u)
            return carry

        lax.fori_loop(0, CH // 2, chunk_body, 0)
        plsc.subcore_barrier()
        pltpu.sync_copy(accum.at[pl.ds(sid * STR, STR)],
                        outp_hbm.at[cid].at[pl.ds(sid * STR, STR)])

    return edge_kernel


_edge1 = _make_edge_kernel(W1ROW, D1)
_edge2 = _make_edge_kernel(W2ROW, D2)


def kernel(x, edge_index, W1, a_src1, a_dst1, b1, W2, a_src2, a_dst2, b2):
    loops = jnp.arange(N, dtype=edge_index.dtype)
    fill = jnp.full((EPAD - ET,), N, edge_index.dtype)
    srcr = jnp.concatenate([edge_index[0], loops, fill]).reshape(EROWS, G)
    dstr = jnp.concatenate([edge_index[1], loops, fill]).reshape(EROWS, G)

    hpad1, asad1, g1 = pl.pallas_call(
        _dense1_body,
        out_shape=[
            jax.ShapeDtypeStruct((NPAD, W1ROW), jnp.float32),
            jax.ShapeDtypeStruct((NPAD, 2), jnp.float32),
            jax.ShapeDtypeStruct((1, 16), jnp.float32),
        ],
    )(x, W1, a_src1.reshape(1, D1), a_dst1.reshape(1, D1))

    outp1 = _edge1(hpad1, asad1[:, 0], asad1[:, 1], g1.reshape(16),
                   srcr, dstr)

    hpad2, asad2, g2 = pl.pallas_call(
        _dense2_body,
        out_shape=[
            jax.ShapeDtypeStruct((NPAD, W2ROW), jnp.float32),
            jax.ShapeDtypeStruct((NPAD, 2), jnp.float32),
            jax.ShapeDtypeStruct((1, 16), jnp.float32),
        ],
    )(outp1, W2, a_src2.reshape(1, D2), a_dst2.reshape(1, D2),
      b1.reshape(1, D1))

    outp2 = _edge2(hpad2, asad2[:, 0], asad2[:, 1], g2.reshape(16),
                   srcr, dstr)

    out = pl.pallas_call(
        _final_body,
        out_shape=jax.ShapeDtypeStruct((N, D2), jnp.float32),
    )(outp2, b2.reshape(1, D2))
    return out


# speedup vs baseline: 1.0642x; 1.0642x over previous
import functools

import jax
import jax.numpy as jnp
from jax import lax
from jax.experimental import pallas as pl
from jax.experimental.pallas import tpu as pltpu
from jax.experimental.pallas import tpu_sc as plsc

N = 10000
E = 320000
ET = E + N
NPAD = 10240
G = 128
NC = 2
NS = 16
NW = NC * NS
CH = 88
EPAD = NW * CH * G
EROWS = EPAD // G
STR = NPAD // NS

D1, W1ROW = 64, 80
D2, W2ROW = 2, 16

_SC_PARAMS = pltpu.CompilerParams(needs_layout_passes=False,
                                  use_tc_tiling_on_sc=False)
_MESH = plsc.VectorSubcoreMesh(core_axis_name="c", subcore_axis_name="s")


def _dense_pack(h, a_src, a_dst, wrow):
    n, d = h.shape
    a_s = jnp.sum(h * a_src, axis=1, keepdims=True)
    a_d = jnp.sum(h * a_dst, axis=1, keepdims=True)
    g = jnp.max(a_s) + jnp.max(a_d)
    g = jnp.where(g >= 0.0, g, 0.2 * g)
    colid = lax.broadcasted_iota(jnp.int32, (n, wrow - d), 1)
    pat = jnp.where(colid == 0, 1.0, 0.0).astype(jnp.float32)
    hp = jnp.concatenate([h, pat], axis=1)
    hp = jnp.concatenate([hp, jnp.zeros((NPAD - n, wrow), jnp.float32)], axis=0)
    asad = jnp.concatenate([a_s, a_d], axis=1)
    asad = jnp.concatenate([asad, jnp.zeros((NPAD - n, 2), jnp.float32)], axis=0)
    return hp, asad, jnp.full((1, 16), g, jnp.float32)


def _dense1_body(x_ref, w_ref, asrc_ref, adst_ref, hpad_ref, asad_ref, g_ref):
    h = jnp.dot(x_ref[:], w_ref[:], preferred_element_type=jnp.float32)
    hp, asad, g = _dense_pack(h, asrc_ref[:], adst_ref[:], W1ROW)
    hpad_ref[:] = hp
    asad_ref[:] = asad
    g_ref[:] = g


def _dense2_body(p_ref, w_ref, asrc_ref, adst_ref, b1_ref,
                 hpad_ref, asad_ref, g_ref):
    p = p_ref[0] + p_ref[1]
    feat = p[0:N, 0:D1]
    den = p[0:N, D1:D1 + 1]
    h1 = feat / (den + 1e-16) + b1_ref[:]
    h1 = jnp.where(h1 > 0.0, h1, jnp.exp(h1) - 1.0)
    h2 = jnp.dot(h1, w_ref[:], preferred_element_type=jnp.float32)
    hp, asad, g = _dense_pack(h2, asrc_ref[:], adst_ref[:], W2ROW)
    hpad_ref[:] = hp
    asad_ref[:] = asad
    g_ref[:] = g


def _final_body(p_ref, b2_ref, out_ref):
    p = p_ref[0] + p_ref[1]
    out_ref[:] = p[0:N, 0:D2] / (p[0:N, D2:D2 + 1] + 1e-16) + b2_ref[:]


def _make_edge_kernel(wrow):
    @functools.partial(
        pl.kernel,
        out_type=jax.ShapeDtypeStruct((NC, NPAD, wrow), jnp.float32),
        mesh=_MESH,
        compiler_params=_SC_PARAMS,
        scratch_types=[
            pltpu.VMEM((NPAD,), jnp.float32),
            pltpu.VMEM((NPAD,), jnp.float32),
            pltpu.VMEM((16,), jnp.float32),
            pltpu.VMEM((CH, G), jnp.int32),
            pltpu.VMEM((CH, G), jnp.int32),
            pltpu.VMEM((G,), jnp.float32),
            [pltpu.VMEM((G, wrow), jnp.float32) for _ in range(2)],
            [pltpu.SemaphoreType.DMA for _ in range(2)],
            pltpu.VMEM_SHARED((NPAD, wrow), jnp.float32),
        ],
    )
    def edge_kernel(hpad_hbm, as_hbm, ad_hbm, g_hbm, src_hbm, dst_hbm,
                    outp_hbm, as_v, ad_v, g_v, src_v, dst_v, ee_v, rows,
                    gsem, accum):
        cid = lax.axis_index("c")
        sid = lax.axis_index("s")
        wid = sid * NC + cid

        def zrow(r, carry):
            for kk in range(wrow // 16):
                rows[0][r, pl.ds(kk * 16, 16)] = jnp.zeros((16,), jnp.float32)
            return carry

        lax.fori_loop(0, G, zrow, 0)
        for i in range(STR // G):
            pltpu.sync_copy(rows[0], accum.at[pl.ds(sid * STR + i * G, G)])

        pltpu.sync_copy(as_hbm, as_v)
        pltpu.sync_copy(ad_hbm, ad_v)
        pltpu.sync_copy(g_hbm, g_v)
        pltpu.sync_copy(src_hbm.at[pl.ds(wid * CH, CH)], src_v)
        pltpu.sync_copy(dst_hbm.at[pl.ds(wid * CH, CH)], dst_v)
        plsc.subcore_barrier()

        gv = g_v[:]

        def gather_issue(c, b):
            pltpu.async_copy(hpad_hbm.at[src_v.at[c]], rows[b], gsem[b])

        def gather_wait(c, b):
            pltpu.make_async_copy(hpad_hbm.at[src_v.at[c]], rows[b],
                                  gsem[b]).wait()

        def process(c, b):
            f = c + 1
            bf = 1 - b

            @pl.when(f < CH)
            def _():
                gather_issue(f, bf)

            for kk in range(G // 16):
                sl = pl.ds(kk * 16, 16)
                e = (plsc.load_gather(as_v, [src_v[c, sl]]) +
                     plsc.load_gather(ad_v, [dst_v[c, sl]]))
                e = jnp.where(e >= 0.0, e, 0.2 * e)
                ee_v[sl] = jnp.exp(e - gv)

            gather_wait(c, b)

            def row_body(i, rcarry):
                for u in range(8):
                    r = i * 8 + u
                    s = plsc.load_gather(ee_v, [jnp.full((16,), r, jnp.int32)])
                    for kk in range(wrow // 16):
                        sl = pl.ds(kk * 16, 16)
                        rows[b][r, sl] = rows[b][r, sl] * s
                return rcarry

            lax.fori_loop(0, G // 8, row_body, 0)
            pltpu.sync_copy(rows[b], accum.at[dst_v.at[c]], add=True)

        gather_issue(0, 0)

        def chunk_body(c2, carry):
            for u in range(2):
                process(c2 * 2 + u, u)
            return carry

        lax.fori_loop(0, CH // 2, chunk_body, 0)
        plsc.subcore_barrier()
        pltpu.sync_copy(accum.at[pl.ds(sid * STR, STR)],
                        outp_hbm.at[cid].at[pl.ds(sid * STR, STR)])

    return edge_kernel


_edge1 = _make_edge_kernel(W1ROW)
_edge2 = _make_edge_kernel(W2ROW)


def kernel(x, edge_index, W1, a_src1, a_dst1, b1, W2, a_src2, a_dst2, b2):
    loops = jnp.arange(N, dtype=edge_index.dtype)
    fill = jnp.full((EPAD - ET,), N, edge_index.dtype)
    srcr = jnp.concatenate([edge_index[0], loops, fill]).reshape(EROWS, G)
    dstr = jnp.concatenate([edge_index[1], loops, fill]).reshape(EROWS, G)

    hpad1, asad1, g1 = pl.pallas_call(
        _dense1_body,
        out_shape=[
            jax.ShapeDtypeStruct((NPAD, W1ROW), jnp.float32),
            jax.ShapeDtypeStruct((NPAD, 2), jnp.float32),
            jax.ShapeDtypeStruct((1, 16), jnp.float32),
        ],
    )(x, W1, a_src1.reshape(1, D1), a_dst1.reshape(1, D1))

    outp1 = _edge1(hpad1, asad1[:, 0], asad1[:, 1], g1.reshape(16),
                   srcr, dstr)

    hpad2, asad2, g2 = pl.pallas_call(
        _dense2_body,
        out_shape=[
            jax.ShapeDtypeStruct((NPAD, W2ROW), jnp.float32),
            jax.ShapeDtypeStruct((NPAD, 2), jnp.float32),
            jax.ShapeDtypeStruct((1, 16), jnp.float32),
        ],
    )(outp1, W2, a_src2.reshape(1, D2), a_dst2.reshape(1, D2),
      b1.reshape(1, D1))

    outp2 = _edge2(hpad2, asad2[:, 0], asad2[:, 1], g2.reshape(16),
                   srcr, dstr)

    out = pl.pallas_call(
        _final_body,
        out_shape=jax.ShapeDtypeStruct((N, D2), jnp.float32),
    )(outp2, b2.reshape(1, D2))
    return out
